# baseline (speedup 1.0000x reference)
import functools

import jax
import jax.numpy as jnp
from jax import lax
from jax.experimental import pallas as pl
from jax.experimental.pallas import tpu as pltpu
from jax.experimental.pallas import tpu_sc as plsc

F = 128
NC, NS = 2, 16
NW = NC * NS
L = 16


def _sc_gather_scale(x_pad, idx0, idx1, n_atoms):
    n_pad = x_pad.shape[0]
    bw = n_pad // NW

    mesh = plsc.VectorSubcoreMesh(core_axis_name="c", subcore_axis_name="s")

    @functools.partial(
        pl.kernel,
        out_type=jax.ShapeDtypeStruct((n_pad, F), jnp.float32),
        mesh=mesh,
        compiler_params=pltpu.CompilerParams(needs_layout_passes=False),
        scratch_types=[
            pltpu.VMEM((bw,), jnp.int32),
            pltpu.VMEM((bw,), jnp.int32),
            pltpu.VMEM((bw,), jnp.int32),
            pltpu.VMEM((bw,), jnp.float32),
            pltpu.VMEM((bw,), jnp.float32),
            pltpu.VMEM((bw, F), jnp.float32),
            pltpu.VMEM((bw, F), jnp.float32),
            pltpu.VMEM((bw, F), jnp.float32),
            pltpu.VMEM((L, L), jnp.float32),
            pltpu.VMEM((L, L), jnp.float32),
            pltpu.VMEM((bw + L,), jnp.float32),
            pltpu.VMEM((bw + L,), jnp.float32),
            pltpu.SemaphoreType.DMA,
            pltpu.SemaphoreType.DMA,
            pltpu.SemaphoreType.DMA,
        ],
    )
    def k(x_hbm, i0_hbm, i1_hbm, g_hbm,
          idxv, safe0, safe1, val0, val1, selfv, nb0, nb1, tr0, tr1,
          cbuf0, cbuf1, sem_s, sem0, sem1):
        wid = lax.axis_index("s") * NC + lax.axis_index("c")
        base = wid * bw

        cp_self = pltpu.async_copy(x_hbm.at[pl.ds(base, bw)], selfv, sem_s)

        def stage_indices(i_hbm, safe_ref, val_ref, sem):
            pltpu.sync_copy(i_hbm.at[pl.ds(base, bw)], idxv)

            def body(j, _):
                iv = idxv[pl.ds(j * L, L)]
                valid = (iv > 0) & (iv < n_atoms)
                safe_ref[pl.ds(j * L, L)] = jnp.clip(iv, 0, n_atoms - 1)
                val_ref[pl.ds(j * L, L)] = jnp.where(valid, 1.0, 0.0)
                return 0

            lax.fori_loop(0, bw // L, body, 0)

        stage_indices(i0_hbm, safe0, val0, sem0)
        cp0 = pltpu.async_copy(x_hbm.at[safe0], nb0, sem0)
        stage_indices(i1_hbm, safe1, val1, sem1)
        cp1 = pltpu.async_copy(x_hbm.at[safe1], nb1, sem1)
        cp_self.wait()
        cp0.wait()
        cp1.wait()

        lane = lax.iota(jnp.int32, L)
        keep = lane >= 3

        def per_group(j, _):
            gbase = j * L
            for t in range(L):
                a = gbase + t
                acc0 = jnp.zeros((L,), jnp.float32)
                acc1 = jnp.zeros((L,), jnp.float32)
                for b in range(F // L):
                    s = selfv[a, pl.ds(b * L, L)]
                    e0 = s - nb0[a, pl.ds(b * L, L)]
                    e1 = s - nb1[a, pl.ds(b * L, L)]
                    if b == 0:
                        e0 = jnp.where(keep, e0, 0.0)
                        e1 = jnp.where(keep, e1, 0.0)
                    acc0 = acc0 + e0 * e0
                    acc1 = acc1 + e1 * e1
                col = jnp.full((L,), t, jnp.int32)
                plsc.store_scatter(tr0, [lane, col], acc0)
                plsc.store_scatter(tr1, [lane, col], acc1)
            d20 = jnp.zeros((L,), jnp.float32)
            d21 = jnp.zeros((L,), jnp.float32)
            for r in range(L):
                d20 = d20 + tr0[r, :]
                d21 = d21 + tr1[r, :]
            c0 = jnp.where(d20 > 0, jnp.where(d20 > 1e-6, 1.0 / d20, 1e6), 1.0)
            c1 = jnp.where(d21 > 0, jnp.where(d21 > 1e-6, 1.0 / d21, 1e6), 1.0)
            cbuf0[pl.ds(gbase, L)] = c0 * val0[pl.ds(gbase, L)]
            cbuf1[pl.ds(gbase, L)] = c1 * val1[pl.ds(gbase, L)]
            return 0

        lax.fori_loop(0, bw // L, per_group, 0)

        def per_group_out(j, _):
            gbase = j * L
            cv0 = cbuf0[pl.ds(gbase, L)]
            cv1 = cbuf1[pl.ds(gbase, L)]
            for t in range(L):
                a = gbase + t
                c0 = cv0[t]
                c1 = cv1[t]
                for b in range(F // L):
                    selfv[a, pl.ds(b * L, L)] = (
                        c0 * nb0[a, pl.ds(b * L, L)]
                        + c1 * nb1[a, pl.ds(b * L, L)]
                    )
            return 0

        lax.fori_loop(0, bw // L, per_group_out, 0)
        pltpu.sync_copy(selfv, g_hbm.at[pl.ds(base, bw)])

    return k(x_pad, idx0, idx1)


def _tc_matmul(x_pad, g, w_s, w_n):
    n_pad = x_pad.shape[0]
    bm = 1024

    def body(x_ref, g_ref, ws_ref, wn_ref, o_ref):
        o_ref[...] = jnp.dot(
            x_ref[...], ws_ref[...], preferred_element_type=jnp.float32
        ) + jnp.dot(g_ref[...], wn_ref[...], preferred_element_type=jnp.float32)

    return pl.pallas_call(
        body,
        grid=(n_pad // bm,),
        in_specs=[
            pl.BlockSpec((bm, F), lambda i: (i, 0)),
            pl.BlockSpec((bm, F), lambda i: (i, 0)),
            pl.BlockSpec((F, F), lambda i: (0, 0)),
            pl.BlockSpec((F, F), lambda i: (0, 0)),
        ],
        out_specs=pl.BlockSpec((bm, F), lambda i: (i, 0)),
        out_shape=jax.ShapeDtypeStruct((n_pad, F), jnp.float32),
    )(x_pad, g, w_s, w_n)


def kernel(inputs, w_s, w_n):
    n = inputs.shape[1]
    n_pad = -(-n // (NW * L)) * (NW * L)
    x = inputs[0, :, :F]
    nbi = inputs[0, :, F:F + 2].astype(jnp.int32)
    x_pad = jnp.pad(x, ((0, n_pad - n), (0, 0)))
    idx0 = jnp.pad(nbi[:, 0], (0, n_pad - n))
    idx1 = jnp.pad(nbi[:, 1], (0, n_pad - n))
    g = _sc_gather_scale(x_pad, idx0, idx1, n)
    out = _tc_matmul(x_pad, g, w_s, w_n)
    return out[:n][None]


# speedup vs baseline: 4.0791x; 4.0791x over previous
import functools

import jax
import jax.numpy as jnp
from jax import lax
from jax.experimental import pallas as pl
from jax.experimental.pallas import tpu as pltpu
from jax.experimental.pallas import tpu_sc as plsc

F = 128
NC, NS = 2, 16
NW = NC * NS
L = 16


def _sc_gather_scale(x_pad, idx0, idx1, n_atoms):
    n_pad = x_pad.shape[0]
    bw = n_pad // NW

    mesh = plsc.VectorSubcoreMesh(core_axis_name="c", subcore_axis_name="s")

    @functools.partial(
        pl.kernel,
        out_type=jax.ShapeDtypeStruct((n_pad, F), jnp.float32),
        mesh=mesh,
        compiler_params=pltpu.CompilerParams(needs_layout_passes=False),
        scratch_types=[
            pltpu.VMEM((bw,), jnp.int32),
            pltpu.VMEM((bw,), jnp.int32),
            pltpu.VMEM((bw,), jnp.int32),
            pltpu.VMEM((bw,), jnp.float32),
            pltpu.VMEM((bw,), jnp.float32),
            pltpu.VMEM((bw, F), jnp.float32),
            pltpu.VMEM((bw, F), jnp.float32),
            pltpu.VMEM((bw, F), jnp.float32),
            pltpu.VMEM((L, L), jnp.float32),
            pltpu.VMEM((L, L), jnp.float32),
            pltpu.VMEM((bw + L,), jnp.float32),
            pltpu.VMEM((bw + L,), jnp.float32),
            pltpu.SemaphoreType.DMA,
            pltpu.SemaphoreType.DMA,
            pltpu.SemaphoreType.DMA,
        ],
    )
    def k(x_hbm, i0_hbm, i1_hbm, g_hbm,
          idxv, safe0, safe1, val0, val1, selfv, nb0, nb1, tr0, tr1,
          cbuf0, cbuf1, sem_s, sem0, sem1):
        wid = lax.axis_index("s") * NC + lax.axis_index("c")
        base = wid * bw

        cp_self = pltpu.async_copy(x_hbm.at[pl.ds(base, bw)], selfv, sem_s)

        lane0 = lax.iota(jnp.int32, L)

        def stage_indices(i_hbm, safe_ref, val_ref, sem):
            pltpu.sync_copy(i_hbm.at[pl.ds(base, bw)], idxv)

            def body(j, _):
                iv = idxv[pl.ds(j * L, L)]
                valid = (iv > 0) & (iv < n_atoms)
                self_idx = base + j * L + lane0
                safe_ref[pl.ds(j * L, L)] = jnp.where(valid, iv, self_idx)
                val_ref[pl.ds(j * L, L)] = jnp.where(valid, 1.0, 0.0)
                return 0

            lax.fori_loop(0, bw // L, body, 0)

        stage_indices(i0_hbm, safe0, val0, sem0)
        cp0 = pltpu.async_copy(x_hbm.at[safe0], nb0, sem0)
        stage_indices(i1_hbm, safe1, val1, sem1)
        cp1 = pltpu.async_copy(x_hbm.at[safe1], nb1, sem1)
        cp_self.wait()
        cp0.wait()
        cp1.wait()

        lane = lax.iota(jnp.int32, L)
        keep = lane >= 3

        def per_group(j, _):
            gbase = j * L
            for t in range(L):
                a = gbase + t
                acc0 = jnp.zeros((L,), jnp.float32)
                acc1 = jnp.zeros((L,), jnp.float32)
                for b in range(F // L):
                    s = selfv[a, pl.ds(b * L, L)]
                    e0 = s - nb0[a, pl.ds(b * L, L)]
                    e1 = s - nb1[a, pl.ds(b * L, L)]
                    if b == 0:
                        e0 = jnp.where(keep, e0, 0.0)
                        e1 = jnp.where(keep, e1, 0.0)
                    acc0 = acc0 + e0 * e0
                    acc1 = acc1 + e1 * e1
                col = jnp.full((L,), t, jnp.int32)
                plsc.store_scatter(tr0, [lane, col], acc0)
                plsc.store_scatter(tr1, [lane, col], acc1)
            d20 = jnp.zeros((L,), jnp.float32)
            d21 = jnp.zeros((L,), jnp.float32)
            for r in range(L):
                d20 = d20 + tr0[r, :]
                d21 = d21 + tr1[r, :]
            c0 = jnp.where(d20 > 0, jnp.where(d20 > 1e-6, 1.0 / d20, 1e6), 1.0)
            c1 = jnp.where(d21 > 0, jnp.where(d21 > 1e-6, 1.0 / d21, 1e6), 1.0)
            cbuf0[pl.ds(gbase, L)] = c0 * val0[pl.ds(gbase, L)]
            cbuf1[pl.ds(gbase, L)] = c1 * val1[pl.ds(gbase, L)]
            return 0

        lax.fori_loop(0, bw // L, per_group, 0)

        def per_group_out(j, _):
            gbase = j * L
            cv0 = cbuf0[pl.ds(gbase, L)]
            cv1 = cbuf1[pl.ds(gbase, L)]
            for t in range(L):
                a = gbase + t
                c0 = cv0[t]
                c1 = cv1[t]
                for b in range(F // L):
                    selfv[a, pl.ds(b * L, L)] = (
                        c0 * nb0[a, pl.ds(b * L, L)]
                        + c1 * nb1[a, pl.ds(b * L, L)]
                    )
            return 0

        lax.fori_loop(0, bw // L, per_group_out, 0)
        pltpu.sync_copy(selfv, g_hbm.at[pl.ds(base, bw)])

    return k(x_pad, idx0, idx1)


def _tc_matmul(x_pad, g, w_s, w_n):
    n_pad = x_pad.shape[0]
    bm = 1024

    def body(x_ref, g_ref, ws_ref, wn_ref, o_ref):
        o_ref[...] = jnp.dot(
            x_ref[...], ws_ref[...], preferred_element_type=jnp.float32
        ) + jnp.dot(g_ref[...], wn_ref[...], preferred_element_type=jnp.float32)

    return pl.pallas_call(
        body,
        grid=(n_pad // bm,),
        in_specs=[
            pl.BlockSpec((bm, F), lambda i: (i, 0)),
            pl.BlockSpec((bm, F), lambda i: (i, 0)),
            pl.BlockSpec((F, F), lambda i: (0, 0)),
            pl.BlockSpec((F, F), lambda i: (0, 0)),
        ],
        out_specs=pl.BlockSpec((bm, F), lambda i: (i, 0)),
        out_shape=jax.ShapeDtypeStruct((n_pad, F), jnp.float32),
    )(x_pad, g, w_s, w_n)


def kernel(inputs, w_s, w_n):
    n = inputs.shape[1]
    n_pad = -(-n // (NW * L)) * (NW * L)
    x = inputs[0, :, :F]
    nbi = inputs[0, :, F:F + 2].astype(jnp.int32)
    x_pad = jnp.pad(x, ((0, n_pad - n), (0, 0)))
    idx0 = jnp.pad(nbi[:, 0], (0, n_pad - n))
    idx1 = jnp.pad(nbi[:, 1], (0, n_pad - n))
    g = _sc_gather_scale(x_pad, idx0, idx1, n)
    out = _tc_matmul(x_pad, g, w_s, w_n)
    return out[:n][None]


# speedup vs baseline: 8.5886x; 2.1055x over previous
import functools

import jax
import jax.numpy as jnp
from jax import lax
from jax.experimental import pallas as pl
from jax.experimental.pallas import tpu as pltpu
from jax.experimental.pallas import tpu_sc as plsc

F = 128
NC, NS = 2, 16
NW = NC * NS
L = 16


CH = 80


def _sc_gather_scale(x_pad, idx0, idx1, n_atoms):
    n_pad = x_pad.shape[0]
    bw = n_pad // NW

    mesh = plsc.VectorSubcoreMesh(core_axis_name="c", subcore_axis_name="s")

    @functools.partial(
        pl.kernel,
        out_type=jax.ShapeDtypeStruct((n_pad, F), jnp.float32),
        mesh=mesh,
        compiler_params=pltpu.CompilerParams(needs_layout_passes=False),
        scratch_types=[
            pltpu.VMEM((bw,), jnp.int32),
            pltpu.VMEM((bw,), jnp.int32),
            pltpu.VMEM((bw,), jnp.int32),
            pltpu.VMEM((bw,), jnp.float32),
            pltpu.VMEM((bw,), jnp.float32),
            pltpu.VMEM((CH, F), jnp.float32),
            pltpu.VMEM((CH, F), jnp.float32),
            pltpu.VMEM((CH, F), jnp.float32),
            pltpu.VMEM((L, L), jnp.float32),
            pltpu.VMEM((L, L), jnp.float32),
            pltpu.VMEM((CH,), jnp.float32),
            pltpu.VMEM((CH,), jnp.float32),
            pltpu.VMEM_SHARED((n_pad, F), jnp.float32),
            pltpu.SemaphoreType.DMA,
            pltpu.SemaphoreType.DMA,
            pltpu.SemaphoreType.DMA,
            pltpu.SemaphoreType.DMA,
        ],
    )
    def k(x_hbm, i0_hbm, i1_hbm, g_hbm,
          idxv, safe0, safe1, val0, val1, selfv, nb0, nb1, tr0, tr1,
          cbuf0, cbuf1, x_sh, sem_s, sem0, sem1, sem_sh):
        sid = lax.axis_index("s")
        wid = sid * NC + lax.axis_index("c")
        base = wid * bw

        sh_rows = n_pad // NS
        cp_sh = pltpu.async_copy(
            x_hbm.at[pl.ds(sid * sh_rows, sh_rows)],
            x_sh.at[pl.ds(sid * sh_rows, sh_rows)],
            sem_sh,
        )

        lane = lax.iota(jnp.int32, L)
        keep = lane >= 3

        def stage_indices(i_hbm, safe_ref, val_ref):
            pltpu.sync_copy(i_hbm.at[pl.ds(base, bw)], idxv)

            def body(j, _):
                iv = idxv[pl.ds(j * L, L)]
                valid = (iv > 0) & (iv < n_atoms)
                self_idx = base + j * L + lane
                safe_ref[pl.ds(j * L, L)] = jnp.where(valid, iv, self_idx)
                val_ref[pl.ds(j * L, L)] = jnp.where(valid, 1.0, 0.0)
                return 0

            lax.fori_loop(0, bw // L, body, 0)

        stage_indices(i0_hbm, safe0, val0)
        stage_indices(i1_hbm, safe1, val1)
        cp_sh.wait()
        plsc.subcore_barrier()

        for c in range(bw // CH):
            cbase = c * CH
            cp_self = pltpu.async_copy(
                x_hbm.at[pl.ds(base + cbase, CH)], selfv, sem_s)
            cp0 = pltpu.async_copy(
                x_sh.at[safe0.at[pl.ds(cbase, CH)]], nb0, sem0)
            cp1 = pltpu.async_copy(
                x_sh.at[safe1.at[pl.ds(cbase, CH)]], nb1, sem1)
            cp_self.wait()
            cp0.wait()
            cp1.wait()

            def per_group(j, _):
                gbase = j * L
                for t in range(L):
                    a = gbase + t
                    acc0 = jnp.zeros((L,), jnp.float32)
                    acc1 = jnp.zeros((L,), jnp.float32)
                    for b in range(F // L):
                        s = selfv[a, pl.ds(b * L, L)]
                        e0 = s - nb0[a, pl.ds(b * L, L)]
                        e1 = s - nb1[a, pl.ds(b * L, L)]
                        if b == 0:
                            e0 = jnp.where(keep, e0, 0.0)
                            e1 = jnp.where(keep, e1, 0.0)
                        acc0 = acc0 + e0 * e0
                        acc1 = acc1 + e1 * e1
                    col = jnp.full((L,), t, jnp.int32)
                    plsc.store_scatter(tr0, [lane, col], acc0)
                    plsc.store_scatter(tr1, [lane, col], acc1)
                d20 = jnp.zeros((L,), jnp.float32)
                d21 = jnp.zeros((L,), jnp.float32)
                for r in range(L):
                    d20 = d20 + tr0[r, :]
                    d21 = d21 + tr1[r, :]
                c0 = jnp.where(d20 > 0, jnp.where(d20 > 1e-6, 1.0 / d20, 1e6), 1.0)
                c1 = jnp.where(d21 > 0, jnp.where(d21 > 1e-6, 1.0 / d21, 1e6), 1.0)
                cbuf0[pl.ds(gbase, L)] = c0 * val0[pl.ds(cbase + gbase, L)]
                cbuf1[pl.ds(gbase, L)] = c1 * val1[pl.ds(cbase + gbase, L)]
                return 0

            lax.fori_loop(0, CH // L, per_group, 0)

            def per_group_out(j, _):
                gbase = j * L
                cv0 = cbuf0[pl.ds(gbase, L)]
                cv1 = cbuf1[pl.ds(gbase, L)]
                for t in range(L):
                    a = gbase + t
                    c0 = cv0[t]
                    c1 = cv1[t]
                    for b in range(F // L):
                        selfv[a, pl.ds(b * L, L)] = (
                            c0 * nb0[a, pl.ds(b * L, L)]
                            + c1 * nb1[a, pl.ds(b * L, L)]
                        )
                return 0

            lax.fori_loop(0, CH // L, per_group_out, 0)
            pltpu.sync_copy(selfv, g_hbm.at[pl.ds(base + cbase, CH)])

    return k(x_pad, idx0, idx1)


def _tc_matmul(x_pad, g, w_s, w_n):
    n_pad = x_pad.shape[0]
    bm = 1024

    def body(x_ref, g_ref, ws_ref, wn_ref, o_ref):
        o_ref[...] = jnp.dot(
            x_ref[...], ws_ref[...], preferred_element_type=jnp.float32
        ) + jnp.dot(g_ref[...], wn_ref[...], preferred_element_type=jnp.float32)

    return pl.pallas_call(
        body,
        grid=(n_pad // bm,),
        in_specs=[
            pl.BlockSpec((bm, F), lambda i: (i, 0)),
            pl.BlockSpec((bm, F), lambda i: (i, 0)),
            pl.BlockSpec((F, F), lambda i: (0, 0)),
            pl.BlockSpec((F, F), lambda i: (0, 0)),
        ],
        out_specs=pl.BlockSpec((bm, F), lambda i: (i, 0)),
        out_shape=jax.ShapeDtypeStruct((n_pad, F), jnp.float32),
    )(x_pad, g, w_s, w_n)


def kernel(inputs, w_s, w_n):
    n = inputs.shape[1]
    n_pad = -(-n // (NW * L)) * (NW * L)
    x = inputs[0, :, :F]
    nbi = inputs[0, :, F:F + 2].astype(jnp.int32)
    x_pad = jnp.pad(x, ((0, n_pad - n), (0, 0)))
    idx0 = jnp.pad(nbi[:, 0], (0, n_pad - n))
    idx1 = jnp.pad(nbi[:, 1], (0, n_pad - n))
    g = _sc_gather_scale(x_pad, idx0, idx1, n)
    out = _tc_matmul(x_pad, g, w_s, w_n)
    return out[:n][None]
